# baseline (speedup 1.0000x reference)
import functools

import jax
import jax.numpy as jnp
from jax import lax
from jax.experimental import pallas as pl
from jax.experimental.pallas import tpu as pltpu
from jax.experimental.pallas import tpu_sc as plsc

N = 10000
E = 320000
D = 128

NC = 2
NS = 16
NW = NC * NS

CHUNK = 128
GCHUNK = 8
GROUP = CHUNK * GCHUNK
NGROUP = 10
EPW = GROUP * NGROUP
E_PAD = EPW * NW

ROWS_PER_TILE = 632
N_PAD = ROWS_PER_TILE * NS


def _sc_mesh():
    return plsc.VectorSubcoreMesh(
        core_axis_name="c", subcore_axis_name="s", num_cores=NC, num_subcores=NS
    )


@functools.partial(
    pl.kernel,
    out_type=jax.ShapeDtypeStruct((NC * N_PAD, D), jnp.float32),
    mesh=_sc_mesh(),
    scratch_types=[
        pltpu.VMEM((2, GCHUNK, CHUNK), jnp.int32),
        pltpu.VMEM((2, GCHUNK, CHUNK), jnp.int32),
        pltpu.VMEM((CHUNK, D), jnp.float32),
        pltpu.VMEM((CHUNK, D), jnp.float32),
        pltpu.VMEM_SHARED((N_PAD, D), jnp.float32),
        pltpu.SemaphoreType.DMA,
        pltpu.SemaphoreType.DMA,
        pltpu.SemaphoreType.DMA,
        pltpu.SemaphoreType.DMA,
    ],
)
def _sc_aggregate(x_hbm, src_hbm, dst_hbm, zeros_hbm, out_hbm,
                  src_ring, dst_ring, rows_a, rows_b, agg_sh,
                  sem_si, sem_di, sem_a, sem_b):
    cid = lax.axis_index("c")
    sid = lax.axis_index("s")
    wid = sid * NC + cid

    my_src = src_hbm.at[wid]
    my_dst = dst_hbm.at[wid]

    pltpu.sync_copy(
        zeros_hbm, agg_sh.at[pl.ds(sid * ROWS_PER_TILE, ROWS_PER_TILE)]
    )

    rows = (rows_a, rows_b)
    sems = (sem_a, sem_b)

    def start_idx_load(g, slot):
        pltpu.async_copy(my_src.at[g], src_ring.at[slot], sem_si)
        pltpu.async_copy(my_dst.at[g], dst_ring.at[slot], sem_di)

    def wait_idx(slot):
        pltpu.make_async_copy(my_src.at[0], src_ring.at[slot], sem_si).wait()
        pltpu.make_async_copy(my_dst.at[0], dst_ring.at[slot], sem_di).wait()

    start_idx_load(0, 0)
    plsc.subcore_barrier()

    def group_body(g, carry):
        slot = lax.rem(g, 2)
        wait_idx(slot)

        @pl.when(g + 1 < NGROUP)
        def _prefetch():
            start_idx_load(g + 1, 1 - slot)

        def gather(k, p):
            pltpu.async_copy(
                x_hbm.at[src_ring.at[slot, k]], rows[p], sems[p]
            )

        def wait_gather(p):
            pltpu.make_async_copy(
                x_hbm.at[pl.ds(0, CHUNK)], rows[p], sems[p]
            ).wait()

        gather(0, 0)
        for k in range(GCHUNK):
            p = k % 2
            if k + 1 < GCHUNK:
                gather(k + 1, 1 - p)
            wait_gather(p)
            pltpu.sync_copy(rows[p], agg_sh.at[dst_ring.at[slot, k]], add=True)
        return carry

    lax.fori_loop(0, NGROUP, group_body, 0)
    plsc.subcore_barrier()

    pltpu.sync_copy(
        agg_sh.at[pl.ds(sid * ROWS_PER_TILE, ROWS_PER_TILE)],
        out_hbm.at[pl.ds(cid * N_PAD + sid * ROWS_PER_TILE, ROWS_PER_TILE)],
    )


TC_BLOCK = 1264
TC_GRID = N_PAD // TC_BLOCK


def _tc_body(p0_ref, p1_ref, w_ref, b_ref, o_ref):
    a = p0_ref[...] + p1_ref[...]
    acc = jnp.dot(a, w_ref[...], preferred_element_type=jnp.float32) + b_ref[...]
    o_ref[...] = acc * jax.nn.sigmoid(acc)


def _tc_finish(partials, W, b2d):
    return pl.pallas_call(
        _tc_body,
        out_shape=jax.ShapeDtypeStruct((N, D), jnp.float32),
        grid=(TC_GRID,),
        in_specs=[
            pl.BlockSpec((TC_BLOCK, D), lambda i: (i, 0)),
            pl.BlockSpec((TC_BLOCK, D), lambda i: (TC_GRID + i, 0)),
            pl.BlockSpec((D, D), lambda i: (0, 0)),
            pl.BlockSpec((1, D), lambda i: (0, 0)),
        ],
        out_specs=pl.BlockSpec((TC_BLOCK, D), lambda i: (i, 0)),
    )(partials, partials, W, b2d)


@jax.jit
def kernel(x, edge_index, W, b):
    pad = E_PAD - E
    src = jnp.concatenate(
        [edge_index[0], jnp.zeros((pad,), jnp.int32)]
    ).reshape(NW, NGROUP, GCHUNK, CHUNK)
    pad_dst = N + jax.lax.rem(jnp.arange(pad, dtype=jnp.int32), N_PAD - N)
    dst = jnp.concatenate(
        [edge_index[1], pad_dst]
    ).reshape(NW, NGROUP, GCHUNK, CHUNK)
    zeros = jnp.zeros((ROWS_PER_TILE, D), jnp.float32)
    partials = _sc_aggregate(x, src, dst, zeros)
    return _tc_finish(partials, W, b.reshape(1, D))


# speedup vs baseline: 2.0001x; 2.0001x over previous
import functools

import jax
import jax.numpy as jnp
from jax import lax
from jax.experimental import pallas as pl
from jax.experimental.pallas import tpu as pltpu
from jax.experimental.pallas import tpu_sc as plsc

N = 10000
E = 320000
D = 128

NC = 2
NS = 16
NW = NC * NS

CHUNK = 96
NCHUNK = 105
EPW = CHUNK * NCHUNK
E_PAD = EPW * NW

ROWS_PER_TILE = 640
N_PAD = ROWS_PER_TILE * NS


def _sc_mesh():
    return plsc.VectorSubcoreMesh(
        core_axis_name="c", subcore_axis_name="s", num_cores=NC, num_subcores=NS
    )


@functools.partial(
    pl.kernel,
    out_type=jax.ShapeDtypeStruct((NC * N_PAD, D), jnp.float32),
    mesh=_sc_mesh(),
    scratch_types=[
        pltpu.VMEM((EPW,), jnp.int32),
        pltpu.VMEM((NCHUNK, CHUNK), jnp.int32),
        pltpu.VMEM((CHUNK, D), jnp.float32),
        pltpu.VMEM((CHUNK, D), jnp.float32),
        pltpu.VMEM_SHARED((N_PAD, D), jnp.float32),
        pltpu.SemaphoreType.DMA,
        pltpu.SemaphoreType.DMA,
    ],
)
def _sc_aggregate(x_hbm, src_hbm, dst_hbm, zeros_hbm, out_hbm,
                  src_v, dst_v, rows_a, rows_b, agg_sh, sem_a, sem_b):
    cid = lax.axis_index("c")
    sid = lax.axis_index("s")
    wid = sid * NC + cid

    pltpu.sync_copy(
        zeros_hbm, agg_sh.at[pl.ds(sid * ROWS_PER_TILE, ROWS_PER_TILE)]
    )

    pltpu.sync_copy(src_hbm.at[pl.ds(wid * EPW, EPW)], src_v)
    pltpu.sync_copy(dst_hbm.at[wid], dst_v)
    plsc.subcore_barrier()

    def start_gather(j, buf, sem):
        idx = src_v.at[pl.ds(j * CHUNK, CHUNK)]
        pltpu.async_copy(x_hbm.at[idx], buf, sem)

    def wait_gather(buf, sem):
        pltpu.make_async_copy(x_hbm.at[pl.ds(0, CHUNK)], buf, sem).wait()

    def scatter(j, buf):
        pltpu.sync_copy(buf, agg_sh.at[dst_v.at[j]], add=True)

    start_gather(0, rows_a, sem_a)

    def body(i, carry):
        start_gather(2 * i + 1, rows_b, sem_b)
        wait_gather(rows_a, sem_a)
        scatter(2 * i, rows_a)
        start_gather(2 * i + 2, rows_a, sem_a)
        wait_gather(rows_b, sem_b)
        scatter(2 * i + 1, rows_b)
        return carry

    lax.fori_loop(0, (NCHUNK - 1) // 2, body, 0)
    wait_gather(rows_a, sem_a)
    scatter(NCHUNK - 1, rows_a)
    plsc.subcore_barrier()

    pltpu.sync_copy(
        agg_sh.at[pl.ds(sid * ROWS_PER_TILE, ROWS_PER_TILE)],
        out_hbm.at[pl.ds(cid * N_PAD + sid * ROWS_PER_TILE, ROWS_PER_TILE)],
    )


TC_BLOCK = 1280
TC_GRID = N_PAD // TC_BLOCK


def _tc_body(p0_ref, p1_ref, w_ref, b_ref, o_ref):
    a = p0_ref[...] + p1_ref[...]
    acc = jnp.dot(a, w_ref[...], preferred_element_type=jnp.float32) + b_ref[...]
    o_ref[...] = acc * jax.nn.sigmoid(acc)


def _tc_finish(partials, W, b2d):
    return pl.pallas_call(
        _tc_body,
        out_shape=jax.ShapeDtypeStruct((N, D), jnp.float32),
        grid=(TC_GRID,),
        in_specs=[
            pl.BlockSpec((TC_BLOCK, D), lambda i: (i, 0)),
            pl.BlockSpec((TC_BLOCK, D), lambda i: (TC_GRID + i, 0)),
            pl.BlockSpec((D, D), lambda i: (0, 0)),
            pl.BlockSpec((1, D), lambda i: (0, 0)),
        ],
        out_specs=pl.BlockSpec((TC_BLOCK, D), lambda i: (i, 0)),
    )(partials, partials, W, b2d)


@jax.jit
def kernel(x, edge_index, W, b):
    pad = E_PAD - E
    src = jnp.concatenate([edge_index[0], jnp.zeros((pad,), jnp.int32)])
    pad_dst = N + jax.lax.rem(jnp.arange(pad, dtype=jnp.int32), N_PAD - N)
    dst = jnp.concatenate([edge_index[1], pad_dst]).reshape(NW, NCHUNK, CHUNK)
    zeros = jnp.zeros((ROWS_PER_TILE, D), jnp.float32)
    partials = _sc_aggregate(x, src, dst, zeros)
    return _tc_finish(partials, W, b.reshape(1, D))


# speedup vs baseline: 2.3722x; 1.1860x over previous
import functools

import jax
import jax.numpy as jnp
from jax import lax
from jax.experimental import pallas as pl
from jax.experimental.pallas import tpu as pltpu
from jax.experimental.pallas import tpu_sc as plsc

N = 10000
E = 320000
D = 128

NC = 2
NS = 16
NW = NC * NS

CHUNK = 64
NCHUNK = 157
EPW = CHUNK * NCHUNK
E_PAD = EPW * NW

ROWS_PER_TILE = 640
N_PAD = ROWS_PER_TILE * NS


def _sc_mesh():
    return plsc.VectorSubcoreMesh(
        core_axis_name="c", subcore_axis_name="s", num_cores=NC, num_subcores=NS
    )


@functools.partial(
    pl.kernel,
    out_type=jax.ShapeDtypeStruct((NC * N_PAD, D), jnp.float32),
    mesh=_sc_mesh(),
    scratch_types=[
        pltpu.VMEM((EPW,), jnp.int32),
        pltpu.VMEM((NCHUNK, CHUNK), jnp.int32),
        pltpu.VMEM((CHUNK, D), jnp.float32),
        pltpu.VMEM((CHUNK, D), jnp.float32),
        pltpu.VMEM_SHARED((N_PAD, D), jnp.float32),
        pltpu.SemaphoreType.DMA,
        pltpu.SemaphoreType.DMA,
    ],
)
def _sc_aggregate(x_hbm, src_hbm, dst_hbm, zeros_hbm, out_hbm,
                  src_v, dst_v, rows_a, rows_b, agg_sh, sem_a, sem_b):
    cid = lax.axis_index("c")
    sid = lax.axis_index("s")
    wid = sid * NC + cid

    pltpu.sync_copy(
        zeros_hbm, agg_sh.at[pl.ds(sid * ROWS_PER_TILE, ROWS_PER_TILE)]
    )

    pltpu.sync_copy(src_hbm.at[pl.ds(wid * EPW, EPW)], src_v)
    pltpu.sync_copy(dst_hbm.at[wid], dst_v)
    plsc.subcore_barrier()

    def start_gather(j, buf, sem):
        idx = src_v.at[pl.ds(j * CHUNK, CHUNK)]
        pltpu.async_copy(x_hbm.at[idx], buf, sem)

    def wait_gather(buf, sem):
        pltpu.make_async_copy(x_hbm.at[pl.ds(0, CHUNK)], buf, sem).wait()

    def scatter(j, buf):
        pltpu.sync_copy(buf, agg_sh.at[dst_v.at[j]], add=True)

    start_gather(0, rows_a, sem_a)

    def body(i, carry):
        start_gather(2 * i + 1, rows_b, sem_b)
        wait_gather(rows_a, sem_a)
        scatter(2 * i, rows_a)
        start_gather(2 * i + 2, rows_a, sem_a)
        wait_gather(rows_b, sem_b)
        scatter(2 * i + 1, rows_b)
        return carry

    lax.fori_loop(0, (NCHUNK - 1) // 2, body, 0)
    wait_gather(rows_a, sem_a)
    scatter(NCHUNK - 1, rows_a)
    plsc.subcore_barrier()

    pltpu.sync_copy(
        agg_sh.at[pl.ds(sid * ROWS_PER_TILE, ROWS_PER_TILE)],
        out_hbm.at[pl.ds(cid * N_PAD + sid * ROWS_PER_TILE, ROWS_PER_TILE)],
    )


TC_BLOCK = 1280
TC_GRID = N_PAD // TC_BLOCK


def _tc_body(p0_ref, p1_ref, w_ref, b_ref, o_ref):
    a = p0_ref[...] + p1_ref[...]
    acc = jnp.dot(a, w_ref[...], preferred_element_type=jnp.float32) + b_ref[...]
    o_ref[...] = acc * jax.nn.sigmoid(acc)


def _tc_finish(partials, W, b2d):
    return pl.pallas_call(
        _tc_body,
        out_shape=jax.ShapeDtypeStruct((N, D), jnp.float32),
        grid=(TC_GRID,),
        in_specs=[
            pl.BlockSpec((TC_BLOCK, D), lambda i: (i, 0)),
            pl.BlockSpec((TC_BLOCK, D), lambda i: (TC_GRID + i, 0)),
            pl.BlockSpec((D, D), lambda i: (0, 0)),
            pl.BlockSpec((1, D), lambda i: (0, 0)),
        ],
        out_specs=pl.BlockSpec((TC_BLOCK, D), lambda i: (i, 0)),
    )(partials, partials, W, b2d)


@jax.jit
def kernel(x, edge_index, W, b):
    pad = E_PAD - E
    src = jnp.concatenate([edge_index[0], jnp.zeros((pad,), jnp.int32)])
    pad_dst = N + jax.lax.rem(jnp.arange(pad, dtype=jnp.int32), N_PAD - N)
    dst = jnp.concatenate([edge_index[1], pad_dst]).reshape(NW, NCHUNK, CHUNK)
    zeros = jnp.zeros((ROWS_PER_TILE, D), jnp.float32)
    partials = _sc_aggregate(x, src, dst, zeros)
    return _tc_finish(partials, W, b.reshape(1, D))


# speedup vs baseline: 3.3782x; 1.4241x over previous
import functools

import jax
import jax.numpy as jnp
from jax import lax
from jax.experimental import pallas as pl
from jax.experimental.pallas import tpu as pltpu
from jax.experimental.pallas import tpu_sc as plsc

N = 10000
E = 320000
D = 128

NC = 2
NS = 16
NW = NC * NS

EPW = E // NW
CHUNK = 80
NCHUNK = EPW // CHUNK
N_PAD = 10240
ROWS_PER_TILE = N_PAD // NS


def _sc_mesh():
    return plsc.VectorSubcoreMesh(
        core_axis_name="c", subcore_axis_name="s", num_cores=NC, num_subcores=NS
    )


@functools.partial(
    pl.kernel,
    out_type=jax.ShapeDtypeStruct((NC * N_PAD, D), jnp.float32),
    mesh=_sc_mesh(),
    scratch_types=[
        pltpu.VMEM((EPW,), jnp.int32),
        pltpu.VMEM((NCHUNK, CHUNK), jnp.int32),
        pltpu.VMEM((CHUNK, D), jnp.float32),
        pltpu.VMEM((CHUNK, D), jnp.float32),
        pltpu.VMEM_SHARED((N_PAD, D), jnp.float32),
        pltpu.SemaphoreType.DMA,
        pltpu.SemaphoreType.DMA,
    ],
)
def _sc_aggregate(x_hbm, src_hbm, dst_hbm, zeros_hbm, out_hbm,
                  src_v, dst_v, rows_a, rows_b, agg_sh, sem_a, sem_b):
    cid = lax.axis_index("c")
    sid = lax.axis_index("s")
    wid = sid * NC + cid

    pltpu.sync_copy(
        zeros_hbm, agg_sh.at[pl.ds(sid * ROWS_PER_TILE, ROWS_PER_TILE)]
    )

    pltpu.sync_copy(src_hbm.at[pl.ds(wid * EPW, EPW)], src_v)
    pltpu.sync_copy(dst_hbm.at[wid], dst_v)
    plsc.subcore_barrier()

    def start_gather(j, buf, sem):
        idx = src_v.at[pl.ds(j * CHUNK, CHUNK)]
        pltpu.async_copy(x_hbm.at[idx], buf, sem)

    def wait_gather(buf, sem):
        pltpu.make_async_copy(x_hbm.at[pl.ds(0, CHUNK)], buf, sem).wait()

    def scatter(j, buf):
        pltpu.sync_copy(buf, agg_sh.at[dst_v.at[j]], add=True)

    start_gather(0, rows_a, sem_a)

    def body(i, carry):
        start_gather(2 * i + 1, rows_b, sem_b)
        wait_gather(rows_a, sem_a)
        scatter(2 * i, rows_a)
        start_gather(2 * i + 2, rows_a, sem_a)
        wait_gather(rows_b, sem_b)
        scatter(2 * i + 1, rows_b)
        return carry

    lax.fori_loop(0, (NCHUNK - 1) // 2, body, 0)
    wait_gather(rows_a, sem_a)
    scatter(NCHUNK - 1, rows_a)
    plsc.subcore_barrier()

    pltpu.sync_copy(
        agg_sh.at[pl.ds(sid * ROWS_PER_TILE, ROWS_PER_TILE)],
        out_hbm.at[pl.ds(cid * N_PAD + sid * ROWS_PER_TILE, ROWS_PER_TILE)],
    )


TC_BLOCK = 1000


def _tc_body(p0_ref, p1_ref, w_ref, b_ref, o_ref):
    a = p0_ref[...] + p1_ref[...]
    acc = jnp.dot(a, w_ref[...], preferred_element_type=jnp.float32) + b_ref[...]
    o_ref[...] = acc * jax.nn.sigmoid(acc)


def _tc_finish(p0, p1, W, b2d):
    return pl.pallas_call(
        _tc_body,
        out_shape=jax.ShapeDtypeStruct((N, D), jnp.float32),
        grid=(N // TC_BLOCK,),
        in_specs=[
            pl.BlockSpec((TC_BLOCK, D), lambda i: (i, 0)),
            pl.BlockSpec((TC_BLOCK, D), lambda i: (i, 0)),
            pl.BlockSpec((D, D), lambda i: (0, 0)),
            pl.BlockSpec((1, D), lambda i: (0, 0)),
        ],
        out_specs=pl.BlockSpec((TC_BLOCK, D), lambda i: (i, 0)),
    )(p0, p1, W, b2d)


@jax.jit
def kernel(x, edge_index, W, b):
    src = edge_index[0]
    dst = edge_index[1].reshape(NW, NCHUNK, CHUNK)
    zeros = jnp.zeros((ROWS_PER_TILE, D), jnp.float32)
    partials = _sc_aggregate(x, src, dst, zeros)
    return _tc_finish(partials[:N], partials[N_PAD:N_PAD + N], W, b.reshape(1, D))


# speedup vs baseline: 3.5469x; 1.0499x over previous
import functools

import jax
import jax.numpy as jnp
from jax import lax
from jax.experimental import pallas as pl
from jax.experimental.pallas import tpu as pltpu
from jax.experimental.pallas import tpu_sc as plsc

N = 10000
E = 320000
D = 128

NC = 2
NS = 16
NW = NC * NS

EPW = E // NW
CHUNK = 80
NCHUNK = EPW // CHUNK
N_PAD = 10240
ROWS_PER_TILE = N_PAD // NS


def _sc_mesh():
    return plsc.VectorSubcoreMesh(
        core_axis_name="c", subcore_axis_name="s", num_cores=NC, num_subcores=NS
    )


@functools.partial(
    pl.kernel,
    out_type=jax.ShapeDtypeStruct((NC * N_PAD, D), jnp.float32),
    mesh=_sc_mesh(),
    scratch_types=[
        pltpu.VMEM((EPW,), jnp.int32),
        pltpu.VMEM((NCHUNK, CHUNK), jnp.int32),
        pltpu.VMEM((CHUNK, D), jnp.float32),
        pltpu.VMEM((CHUNK, D), jnp.float32),
        pltpu.VMEM_SHARED((N_PAD, D), jnp.float32),
        pltpu.SemaphoreType.DMA,
        pltpu.SemaphoreType.DMA,
    ],
)
def _sc_aggregate(x_hbm, src_hbm, dst_hbm, zeros_hbm, out_hbm,
                  src_v, dst_v, rows_a, rows_b, agg_sh, sem_a, sem_b):
    cid = lax.axis_index("c")
    sid = lax.axis_index("s")
    wid = sid * NC + cid

    pltpu.sync_copy(
        zeros_hbm, agg_sh.at[pl.ds(sid * ROWS_PER_TILE, ROWS_PER_TILE)]
    )

    pltpu.sync_copy(src_hbm.at[pl.ds(wid * EPW, EPW)], src_v)
    pltpu.sync_copy(dst_hbm.at[wid], dst_v)
    plsc.subcore_barrier()

    def start_gather(j, buf, sem):
        idx = src_v.at[pl.ds(j * CHUNK, CHUNK)]
        pltpu.async_copy(x_hbm.at[idx], buf, sem)

    def wait_gather(buf, sem):
        pltpu.make_async_copy(x_hbm.at[pl.ds(0, CHUNK)], buf, sem).wait()

    def scatter(j, buf):
        pltpu.sync_copy(buf, agg_sh.at[dst_v.at[j]], add=True)

    start_gather(0, rows_a, sem_a)

    def body(i, carry):
        start_gather(2 * i + 1, rows_b, sem_b)
        wait_gather(rows_a, sem_a)
        scatter(2 * i, rows_a)
        start_gather(2 * i + 2, rows_a, sem_a)
        wait_gather(rows_b, sem_b)
        scatter(2 * i + 1, rows_b)
        return carry

    lax.fori_loop(0, (NCHUNK - 1) // 2, body, 0)
    wait_gather(rows_a, sem_a)
    scatter(NCHUNK - 1, rows_a)
    plsc.subcore_barrier()

    pltpu.sync_copy(
        agg_sh.at[pl.ds(sid * ROWS_PER_TILE, ROWS_PER_TILE)],
        out_hbm.at[pl.ds(cid * N_PAD + sid * ROWS_PER_TILE, ROWS_PER_TILE)],
    )


TC_BLOCK = 1280
TC_GRID = N_PAD // TC_BLOCK


def _tc_body(p0_ref, p1_ref, w_ref, b_ref, o_ref):
    a = p0_ref[...] + p1_ref[...]
    acc = jnp.dot(a, w_ref[...], preferred_element_type=jnp.float32) + b_ref[...]
    o_ref[...] = acc * jax.nn.sigmoid(acc)


def _tc_finish(partials, W, b2d):
    return pl.pallas_call(
        _tc_body,
        out_shape=jax.ShapeDtypeStruct((N, D), jnp.float32),
        grid=(TC_GRID,),
        in_specs=[
            pl.BlockSpec((TC_BLOCK, D), lambda i: (i, 0)),
            pl.BlockSpec((TC_BLOCK, D), lambda i: (TC_GRID + i, 0)),
            pl.BlockSpec((D, D), lambda i: (0, 0)),
            pl.BlockSpec((1, D), lambda i: (0, 0)),
        ],
        out_specs=pl.BlockSpec((TC_BLOCK, D), lambda i: (i, 0)),
    )(partials, partials, W, b2d)


@jax.jit
def kernel(x, edge_index, W, b):
    src = edge_index[0]
    dst = edge_index[1].reshape(NW, NCHUNK, CHUNK)
    zeros = jnp.zeros((ROWS_PER_TILE, D), jnp.float32)
    partials = _sc_aggregate(x, src, dst, zeros)
    return _tc_finish(partials, W, b.reshape(1, D))
